# baseline (speedup 1.0000x reference)
import functools

import jax
import jax.numpy as jnp
from jax.experimental import pallas as pl
from jax.experimental.pallas import tpu as pltpu
from jax.experimental.pallas import tpu_sc as plsc

D = 75
DP = 128
KNN = 14
QBLK = 256


def _prep_body(k_ref, kT_ref, kn_ref, kTn_ref, k2_ref):
    KTOT = k_ref.shape[0]

    k = (k_ref[...] - 0.5) * 2.0
    lane = jax.lax.broadcasted_iota(jnp.int32, (KTOT, DP), 1)
    kmean = jnp.sum(k, axis=1, keepdims=True) / float(D)
    kn_ref[...] = jnp.where(lane < D, k - kmean, 0.0)

    kT = (kT_ref[...] - 0.5) * 2.0
    sub = jax.lax.broadcasted_iota(jnp.int32, (DP, KTOT), 0)
    kmeanT = jnp.sum(kT, axis=0, keepdims=True) / float(D)
    kTn = jnp.where(sub < D, kT - kmeanT, 0.0)
    k2 = jnp.sum(kTn * kTn, axis=0, keepdims=True)
    kTn_ref[...] = kTn.astype(jnp.bfloat16)
    k2_ref[...] = jnp.broadcast_to(k2, (8, KTOT))


def _run_prep(kp, kTp):
    K = kp.shape[0]
    return pl.pallas_call(
        _prep_body,
        in_specs=[
            pl.BlockSpec((K, DP), lambda: (0, 0)),
            pl.BlockSpec((DP, K), lambda: (0, 0)),
        ],
        out_specs=[
            pl.BlockSpec((K, DP), lambda: (0, 0)),
            pl.BlockSpec((DP, K), lambda: (0, 0)),
            pl.BlockSpec((8, K), lambda: (0, 0)),
        ],
        out_shape=[
            jax.ShapeDtypeStruct((K, DP), jnp.float32),
            jax.ShapeDtypeStruct((DP, K), jnp.bfloat16),
            jax.ShapeDtypeStruct((8, K), jnp.float32),
        ],
    )(kp, kTp)


def _main_body(q_ref, kTn_ref, k2_ref, ind_ref, w_ref):
    f32 = jnp.float32

    q = (q_ref[...] - 0.5) * 2.0
    lane_q = jax.lax.broadcasted_iota(jnp.int32, (QBLK, DP), 1)
    qmean = jnp.sum(q, axis=1, keepdims=True) / float(D)
    qn = jnp.where(lane_q < D, q - qmean, 0.0)
    q2 = jnp.sum(qn * qn, axis=1, keepdims=True)

    kTn = kTn_ref[...]
    KTOT = kTn.shape[1]
    k2 = k2_ref[0:1, :]

    t = jax.lax.dot_general(qn.astype(jnp.bfloat16), kTn,
                            (((1,), (0,)), ((), ())),
                            preferred_element_type=f32)
    nd = -((q2 + k2) - 2.0 * t)

    lane_out = jax.lax.broadcasted_iota(jnp.int32, (QBLK, DP), 1)

    HQ = KTOT // 4
    lane4 = jax.lax.broadcasted_iota(jnp.int32, (QBLK, HQ), 1)
    v0 = nd[:, :HQ]
    v1 = nd[:, HQ : 2 * HQ]
    v2 = nd[:, 2 * HQ : 3 * HQ]
    v3 = nd[:, 3 * HQ :]
    i0 = lane4
    i1 = lane4 + HQ
    i2 = lane4 + 2 * HQ
    i3 = lane4 + 3 * HQ

    gA = v0 >= v2
    Va, Ia = jnp.where(gA, v0, v2), jnp.where(gA, i0, i2)
    La, Ja = jnp.where(gA, v2, v0), jnp.where(gA, i2, i0)
    gB = v1 >= v3
    Vb, Ib = jnp.where(gB, v1, v3), jnp.where(gB, i1, i3)
    Lb, Jb = jnp.where(gB, v3, v1), jnp.where(gB, i3, i1)

    def _lexge(va, ia, vb, ib):
        return (va > vb) | ((va == vb) & (ia < ib))

    c1 = _lexge(Va, Ia, Vb, Ib)
    V1, I1 = jnp.where(c1, Va, Vb), jnp.where(c1, Ia, Ib)
    Xv, Xi = jnp.where(c1, Vb, Va), jnp.where(c1, Ib, Ia)
    c2 = _lexge(La, Ja, Lb, Jb)
    V4, I4 = jnp.where(c2, Lb, La), jnp.where(c2, Jb, Ja)
    Yv, Yi = jnp.where(c2, La, Lb), jnp.where(c2, Ja, Jb)
    c3 = _lexge(Xv, Xi, Yv, Yi)
    V2, I2 = jnp.where(c3, Xv, Yv), jnp.where(c3, Xi, Yi)
    V3, I3 = jnp.where(c3, Yv, Xv), jnp.where(c3, Yi, Xi)

    inds = jnp.zeros((QBLK, DP), jnp.int32)
    w = jnp.zeros((QBLK, DP), f32)
    wsum = jnp.zeros((QBLK, 1), f32)
    m0 = None
    for j in range(KNN):
        m = jnp.max(V1, axis=1, keepdims=True)
        idx = jnp.min(jnp.where(V1 == m, I1, KTOT),
                      axis=1, keepdims=True)
        if j == 0:
            m0 = m
            e = jnp.ones((QBLK, 1), f32)
        else:
            e = jnp.exp(m - m0)
        inds = jnp.where(lane_out == j, idx, inds)
        w = jnp.where(lane_out == j, e, w)
        wsum = wsum + e
        hit = I1 == idx
        V1 = jnp.where(hit, V2, V1)
        I1 = jnp.where(hit, I2, I1)
        V2 = jnp.where(hit, V3, V2)
        I2 = jnp.where(hit, I3, I2)
        V3 = jnp.where(hit, V4, V3)
        I3 = jnp.where(hit, I4, I3)
        V4 = jnp.where(hit, -jnp.inf, V4)

    ind_ref[...] = inds
    w_ref[...] = w / wsum


def _run_main(qp, kTn, k2b):
    Q = qp.shape[0]
    K = kTn.shape[1]
    return pl.pallas_call(
        _main_body,
        grid=(Q // QBLK,),
        in_specs=[
            pl.BlockSpec((QBLK, DP), lambda i: (i, 0)),
            pl.BlockSpec((DP, K), lambda i: (0, 0)),
            pl.BlockSpec((8, K), lambda i: (0, 0)),
        ],
        out_specs=[
            pl.BlockSpec((QBLK, DP), lambda i: (i, 0)),
            pl.BlockSpec((QBLK, DP), lambda i: (i, 0)),
        ],
        out_shape=[
            jax.ShapeDtypeStruct((Q, DP), jnp.int32),
            jax.ShapeDtypeStruct((Q, DP), jnp.float32),
        ],
    )(qp, kTn, k2b)


GWIN = 128


def _run_sc_gather(kn, idx_flat):
    B = idx_flat.shape[0]
    idx2 = idx_flat.reshape(1, B)
    mesh = plsc.VectorSubcoreMesh(core_axis_name="c", subcore_axis_name="s")

    @functools.partial(
        pl.kernel,
        out_type=jax.ShapeDtypeStruct((B, DP), jnp.float32),
        mesh=mesh,
    )
    def _sc_kernel(kn_hbm, idx_hbm, out_hbm):
        def body(i_vmem, o_vmem):
            pltpu.sync_copy(kn_hbm.at[i_vmem.at[0]], o_vmem)

        pltpu.emit_pipeline(
            body,
            grid=(B // GWIN,),
            in_specs=[pl.BlockSpec((1, GWIN), index_map=lambda i: (0, i))],
            out_specs=[pl.BlockSpec((GWIN, DP), index_map=lambda i: (i, 0))],
            core_axis_name=("c", "s"),
            dimension_semantics=(pltpu.PARALLEL,),
        )(idx_hbm, out_hbm)

    return _sc_kernel(kn, idx2)


def _reduce_body(nbr_ref, w_ref, agg_ref):
    acc = nbr_ref[0] * w_ref[:, 0:1]
    for j in range(1, KNN):
        acc = acc + nbr_ref[j] * w_ref[:, j : j + 1]
    agg_ref[...] = acc


def _run_reduce(nbr_slabs, w):
    Q = w.shape[0]
    return pl.pallas_call(
        _reduce_body,
        grid=(Q // QBLK,),
        in_specs=[
            pl.BlockSpec((KNN, QBLK, DP), lambda i: (0, i, 0)),
            pl.BlockSpec((QBLK, DP), lambda i: (i, 0)),
        ],
        out_specs=pl.BlockSpec((QBLK, DP), lambda i: (i, 0)),
        out_shape=jax.ShapeDtypeStruct((Q, DP), jnp.float32),
    )(nbr_slabs, w)


def kernel(queries, keys):
    Q = queries.shape[0]
    K = keys.shape[0]
    qp = jnp.pad(queries, ((0, 0), (0, DP - D)), constant_values=0.5)
    kp = jnp.pad(keys, ((0, 0), (0, DP - D)), constant_values=0.5)
    kTp = jnp.pad(keys.T, ((0, DP - D), (0, 0)), constant_values=0.5)

    kn, kTn, k2b = _run_prep(kp, kTp)
    inds_pad, w_pad = _run_main(qp, kTn, k2b)
    inds = inds_pad[:, :KNN]

    idx_flat = inds.T.reshape(KNN * Q)
    nbr = _run_sc_gather(kn, idx_flat)
    nbr_slabs = nbr.reshape(KNN, Q, DP)

    agg = _run_reduce(nbr_slabs, w_pad)
    return agg[:, :D], inds


# speedup vs baseline: 1.1323x; 1.1323x over previous
import functools

import jax
import jax.numpy as jnp
from jax.experimental import pallas as pl
from jax.experimental.pallas import tpu as pltpu
from jax.experimental.pallas import tpu_sc as plsc

D = 75
DP = 128
KNN = 14
QBLK = 128


def _prep_body(k_ref, kT_ref, kn_ref, kTn_ref, k2_ref):
    KTOT = k_ref.shape[0]

    k = (k_ref[...] - 0.5) * 2.0
    lane = jax.lax.broadcasted_iota(jnp.int32, (KTOT, DP), 1)
    kmean = jnp.sum(k, axis=1, keepdims=True) / float(D)
    kn_ref[...] = jnp.where(lane < D, k - kmean, 0.0)

    kT = (kT_ref[...] - 0.5) * 2.0
    sub = jax.lax.broadcasted_iota(jnp.int32, (DP, KTOT), 0)
    kmeanT = jnp.sum(kT, axis=0, keepdims=True) / float(D)
    kTn = jnp.where(sub < D, kT - kmeanT, 0.0)
    k2 = jnp.sum(kTn * kTn, axis=0, keepdims=True)
    kTn_ref[...] = kTn.astype(jnp.bfloat16)
    k2_ref[...] = jnp.broadcast_to(k2, (8, KTOT))


def _run_prep(kp, kTp):
    K = kp.shape[0]
    return pl.pallas_call(
        _prep_body,
        in_specs=[
            pl.BlockSpec((K, DP), lambda: (0, 0)),
            pl.BlockSpec((DP, K), lambda: (0, 0)),
        ],
        out_specs=[
            pl.BlockSpec((K, DP), lambda: (0, 0)),
            pl.BlockSpec((DP, K), lambda: (0, 0)),
            pl.BlockSpec((8, K), lambda: (0, 0)),
        ],
        out_shape=[
            jax.ShapeDtypeStruct((K, DP), jnp.float32),
            jax.ShapeDtypeStruct((DP, K), jnp.bfloat16),
            jax.ShapeDtypeStruct((8, K), jnp.float32),
        ],
    )(kp, kTp)


def _main_body(q_ref, kTn_ref, k2_ref, ind_ref, w_ref):
    f32 = jnp.float32

    q = (q_ref[...] - 0.5) * 2.0
    lane_q = jax.lax.broadcasted_iota(jnp.int32, (QBLK, DP), 1)
    qmean = jnp.sum(q, axis=1, keepdims=True) / float(D)
    qn = jnp.where(lane_q < D, q - qmean, 0.0)
    q2 = jnp.sum(qn * qn, axis=1, keepdims=True)

    kTn = kTn_ref[...]
    KTOT = kTn.shape[1]
    k2 = k2_ref[0:1, :]

    t = jax.lax.dot_general(qn.astype(jnp.bfloat16), kTn,
                            (((1,), (0,)), ((), ())),
                            preferred_element_type=f32)
    nd = -((q2 + k2) - 2.0 * t)

    lane_out = jax.lax.broadcasted_iota(jnp.int32, (QBLK, DP), 1)

    HQ = KTOT // 4
    lane4 = jax.lax.broadcasted_iota(jnp.int32, (QBLK, HQ), 1)
    v0 = nd[:, :HQ]
    v1 = nd[:, HQ : 2 * HQ]
    v2 = nd[:, 2 * HQ : 3 * HQ]
    v3 = nd[:, 3 * HQ :]
    i0 = lane4
    i1 = lane4 + HQ
    i2 = lane4 + 2 * HQ
    i3 = lane4 + 3 * HQ

    gA = v0 >= v2
    Va, Ia = jnp.where(gA, v0, v2), jnp.where(gA, i0, i2)
    La, Ja = jnp.where(gA, v2, v0), jnp.where(gA, i2, i0)
    gB = v1 >= v3
    Vb, Ib = jnp.where(gB, v1, v3), jnp.where(gB, i1, i3)
    Lb, Jb = jnp.where(gB, v3, v1), jnp.where(gB, i3, i1)

    def _lexge(va, ia, vb, ib):
        return (va > vb) | ((va == vb) & (ia < ib))

    c1 = _lexge(Va, Ia, Vb, Ib)
    V1, I1 = jnp.where(c1, Va, Vb), jnp.where(c1, Ia, Ib)
    Xv, Xi = jnp.where(c1, Vb, Va), jnp.where(c1, Ib, Ia)
    c2 = _lexge(La, Ja, Lb, Jb)
    V4, I4 = jnp.where(c2, Lb, La), jnp.where(c2, Jb, Ja)
    Yv, Yi = jnp.where(c2, La, Lb), jnp.where(c2, Ja, Jb)
    c3 = _lexge(Xv, Xi, Yv, Yi)
    V2, I2 = jnp.where(c3, Xv, Yv), jnp.where(c3, Xi, Yi)
    V3, I3 = jnp.where(c3, Yv, Xv), jnp.where(c3, Yi, Xi)

    inds = jnp.zeros((QBLK, DP), jnp.int32)
    w = jnp.zeros((QBLK, DP), f32)
    wsum = jnp.zeros((QBLK, 1), f32)
    m0 = None
    for j in range(KNN):
        m = jnp.max(V1, axis=1, keepdims=True)
        idx = jnp.min(jnp.where(V1 == m, I1, KTOT),
                      axis=1, keepdims=True)
        if j == 0:
            m0 = m
            e = jnp.ones((QBLK, 1), f32)
        else:
            e = jnp.exp(m - m0)
        inds = jnp.where(lane_out == j, idx, inds)
        w = jnp.where(lane_out == j, e, w)
        wsum = wsum + e
        hit = I1 == idx
        V1 = jnp.where(hit, V2, V1)
        I1 = jnp.where(hit, I2, I1)
        V2 = jnp.where(hit, V3, V2)
        I2 = jnp.where(hit, I3, I2)
        V3 = jnp.where(hit, V4, V3)
        I3 = jnp.where(hit, I4, I3)
        V4 = jnp.where(hit, -jnp.inf, V4)

    ind_ref[...] = inds
    w_ref[...] = w / wsum


def _run_main(qp, kTn, k2b):
    Q = qp.shape[0]
    K = kTn.shape[1]
    return pl.pallas_call(
        _main_body,
        grid=(Q // QBLK,),
        in_specs=[
            pl.BlockSpec((QBLK, DP), lambda i: (i, 0)),
            pl.BlockSpec((DP, K), lambda i: (0, 0)),
            pl.BlockSpec((8, K), lambda i: (0, 0)),
        ],
        out_specs=[
            pl.BlockSpec((QBLK, DP), lambda i: (i, 0)),
            pl.BlockSpec((QBLK, DP), lambda i: (i, 0)),
        ],
        out_shape=[
            jax.ShapeDtypeStruct((Q, DP), jnp.int32),
            jax.ShapeDtypeStruct((Q, DP), jnp.float32),
        ],
    )(qp, kTn, k2b)


GWIN = 128


def _run_sc_gather(kn, idx_flat):
    B = idx_flat.shape[0]
    idx2 = idx_flat.reshape(1, B)
    mesh = plsc.VectorSubcoreMesh(core_axis_name="c", subcore_axis_name="s")

    @functools.partial(
        pl.kernel,
        out_type=jax.ShapeDtypeStruct((B, DP), jnp.float32),
        mesh=mesh,
    )
    def _sc_kernel(kn_hbm, idx_hbm, out_hbm):
        def body(i_vmem, o_vmem):
            pltpu.sync_copy(kn_hbm.at[i_vmem.at[0]], o_vmem)

        pltpu.emit_pipeline(
            body,
            grid=(B // GWIN,),
            in_specs=[pl.BlockSpec((1, GWIN), index_map=lambda i: (0, i))],
            out_specs=[pl.BlockSpec((GWIN, DP), index_map=lambda i: (i, 0))],
            core_axis_name=("c", "s"),
            dimension_semantics=(pltpu.PARALLEL,),
        )(idx_hbm, out_hbm)

    return _sc_kernel(kn, idx2)


def _reduce_body(nbr_ref, w_ref, agg_ref):
    acc = nbr_ref[0] * w_ref[:, 0:1]
    for j in range(1, KNN):
        acc = acc + nbr_ref[j] * w_ref[:, j : j + 1]
    agg_ref[...] = acc


def _run_reduce(nbr_slabs, w):
    Q = w.shape[0]
    return pl.pallas_call(
        _reduce_body,
        grid=(Q // QBLK,),
        in_specs=[
            pl.BlockSpec((KNN, QBLK, DP), lambda i: (0, i, 0)),
            pl.BlockSpec((QBLK, DP), lambda i: (i, 0)),
        ],
        out_specs=pl.BlockSpec((QBLK, DP), lambda i: (i, 0)),
        out_shape=jax.ShapeDtypeStruct((Q, DP), jnp.float32),
    )(nbr_slabs, w)


def kernel(queries, keys):
    Q = queries.shape[0]
    K = keys.shape[0]
    qp = jnp.pad(queries, ((0, 0), (0, DP - D)), constant_values=0.5)
    kp = jnp.pad(keys, ((0, 0), (0, DP - D)), constant_values=0.5)
    kTp = jnp.pad(keys.T, ((0, DP - D), (0, 0)), constant_values=0.5)

    kn, kTn, k2b = _run_prep(kp, kTp)
    inds_pad, w_pad = _run_main(qp, kTn, k2b)
    inds = inds_pad[:, :KNN]

    idx_flat = inds.T.reshape(KNN * Q)
    nbr = _run_sc_gather(kn, idx_flat)
    nbr_slabs = nbr.reshape(KNN, Q, DP)

    agg = _run_reduce(nbr_slabs, w_pad)
    return agg[:, :D], inds
